# Initial kernel scaffold; baseline (speedup 1.0000x reference)
import jax
import jax.numpy as jnp
from jax.experimental import pallas as pl


def kernel(x, edge_index, W1, b1, W2, b2):
    raise NotImplementedError("write your pallas kernel here")


# speedup vs baseline: 19.2394x; 19.2394x over previous
import jax
import jax.numpy as jnp
from jax import lax
from jax.experimental import pallas as pl
from jax.experimental.pallas import tpu as pltpu
from jax.experimental.pallas import tpu_sc as plsc

N = 10000
E = 320000
D = 128

NC = 2
NS = 16
NW = NC * NS
K = 80
STEPS = E // (NW * K)
NP = 10240
RPT = NP // NS

_MESH = plsc.VectorSubcoreMesh(core_axis_name="c", subcore_axis_name="s")


def _cnt_body(dst3_hbm, ones_hbm, z16_hbm, out_hbm, dst_v, ones_v, cacc):
    c = lax.axis_index("c")
    s = lax.axis_index("s")
    wid = c * NS + s
    pltpu.sync_copy(z16_hbm, cacc.at[pl.ds(s * RPT, RPT)])
    pltpu.sync_copy(ones_hbm, ones_v)
    pltpu.sync_copy(dst3_hbm.at[wid], dst_v)
    plsc.subcore_barrier()

    def step(j, carry):
        pltpu.sync_copy(ones_v, cacc.at[dst_v.at[j]], add=True)
        return carry

    lax.fori_loop(0, STEPS, step, 0)
    plsc.subcore_barrier()
    pltpu.sync_copy(cacc.at[pl.ds(s * RPT, RPT)],
                    out_hbm.at[c, pl.ds(s * RPT, RPT)])


_cnt_kernel = pl.kernel(
    _cnt_body,
    out_type=jax.ShapeDtypeStruct((NC, NP, 16), jnp.float32),
    mesh=_MESH,
    scratch_types=[
        pltpu.VMEM((STEPS, K), jnp.int32),
        pltpu.VMEM((K, 16), jnp.float32),
        pltpu.VMEM_SHARED((NP, 16), jnp.float32),
    ],
    compiler_params=pltpu.CompilerParams(use_tc_tiling_on_sc=False),
)


def _edge_body(y_hbm, src3_hbm, dst3_hbm, zrows_hbm, out_hbm, src_v, dst_v,
               rows_v, zacc, sem):
    c = lax.axis_index("c")
    s = lax.axis_index("s")
    wid = c * NS + s
    pltpu.sync_copy(zrows_hbm, zacc.at[pl.ds(s * RPT, RPT)])
    pltpu.sync_copy(src3_hbm.at[wid], src_v)
    pltpu.sync_copy(dst3_hbm.at[wid], dst_v)
    plsc.subcore_barrier()

    def step(j, carry):
        pltpu.async_copy(y_hbm.at[src_v.at[j]], rows_v, sem).wait()
        pltpu.sync_copy(rows_v, zacc.at[dst_v.at[j]], add=True)
        return carry

    lax.fori_loop(0, STEPS, step, 0)
    plsc.subcore_barrier()
    pltpu.sync_copy(zacc.at[pl.ds(s * RPT, RPT)],
                    out_hbm.at[c, pl.ds(s * RPT, RPT)])


_edge_kernel = pl.kernel(
    _edge_body,
    out_type=jax.ShapeDtypeStruct((NC, NP, D), jnp.float32),
    mesh=_MESH,
    scratch_types=[
        pltpu.VMEM((STEPS, K), jnp.int32),
        pltpu.VMEM((STEPS, K), jnp.int32),
        pltpu.VMEM((K, D), jnp.float32),
        pltpu.VMEM_SHARED((NP, D), jnp.float32),
        pltpu.SemaphoreType.DMA,
    ],
)


BN = 1000


def _dinv(cnt_ref):
    deg = cnt_ref[0, :, 0:1] + cnt_ref[1, :, 0:1] + 1.0
    return lax.rsqrt(deg)


def _k1_body(cnt_ref, x_ref, w_ref, y_ref):
    y_ref[...] = _dinv(cnt_ref) * jnp.dot(
        x_ref[...], w_ref[...], preferred_element_type=jnp.float32)


def _k2_body(cnt_ref, z_ref, y_ref, b_ref, w_ref, o_ref):
    dinv = _dinv(cnt_ref)
    h = jnp.maximum(
        dinv * (z_ref[0] + z_ref[1] + y_ref[...]) + b_ref[...], 0.0)
    o_ref[...] = dinv * jnp.dot(
        h, w_ref[...], preferred_element_type=jnp.float32)


def _k3_body(cnt_ref, z_ref, y_ref, b_ref, o_ref):
    o_ref[...] = (_dinv(cnt_ref) * (z_ref[0] + z_ref[1] + y_ref[...])
                  + b_ref[...])


_cnt_spec = pl.BlockSpec((NC, BN, 16), lambda i: (0, i, 0))
_row_spec = pl.BlockSpec((BN, D), lambda i: (i, 0))
_z_spec = pl.BlockSpec((NC, BN, D), lambda i: (0, i, 0))
_w_spec = pl.BlockSpec((D, D), lambda i: (0, 0))
_b_spec = pl.BlockSpec((1, D), lambda i: (0, 0))
_out_shape = jax.ShapeDtypeStruct((N, D), jnp.float32)

_k1 = pl.pallas_call(
    _k1_body, grid=(N // BN,),
    in_specs=[_cnt_spec, _row_spec, _w_spec],
    out_specs=_row_spec, out_shape=_out_shape)

_k2 = pl.pallas_call(
    _k2_body, grid=(N // BN,),
    in_specs=[_cnt_spec, _z_spec, _row_spec, _b_spec, _w_spec],
    out_specs=_row_spec, out_shape=_out_shape)

_k3 = pl.pallas_call(
    _k3_body, grid=(N // BN,),
    in_specs=[_cnt_spec, _z_spec, _row_spec, _b_spec],
    out_specs=_row_spec, out_shape=_out_shape)


def kernel(x, edge_index, W1, b1, W2, b2):
    src3 = edge_index[0].reshape(NW, STEPS, K)
    dst3 = edge_index[1].reshape(NW, STEPS, K)
    ones16 = jnp.ones((K, 16), jnp.float32)
    z16 = jnp.zeros((RPT, 16), jnp.float32)
    zrows = jnp.zeros((RPT, D), jnp.float32)

    cnt = _cnt_kernel(dst3, ones16, z16)
    y1 = _k1(cnt, x, W1)
    z1 = _edge_kernel(y1, src3, dst3, zrows)
    y2 = _k2(cnt, z1, y1, b1.reshape(1, D), W2)
    z2 = _edge_kernel(y2, src3, dst3, zrows)
    return _k3(cnt, z2, y2, b2.reshape(1, D))


# speedup vs baseline: 29.1654x; 1.5159x over previous
import jax
import jax.numpy as jnp
from jax import lax
from jax.experimental import pallas as pl
from jax.experimental.pallas import tpu as pltpu
from jax.experimental.pallas import tpu_sc as plsc

N = 10000
E = 320000
D = 128

NC = 2
NS = 16
NW = NC * NS
K = 80
STEPS = E // (NW * K)
NP = 10240
RPT = NP // NS

_MESH = plsc.VectorSubcoreMesh(core_axis_name="c", subcore_axis_name="s")


def _cnt_body(dst3_hbm, ones_hbm, z16_hbm, out_hbm, dst_v, ones_v, cacc):
    c = lax.axis_index("c")
    s = lax.axis_index("s")
    wid = c * NS + s
    pltpu.sync_copy(z16_hbm, cacc.at[pl.ds(s * RPT, RPT)])
    pltpu.sync_copy(ones_hbm, ones_v)
    pltpu.sync_copy(dst3_hbm.at[wid], dst_v)
    plsc.subcore_barrier()

    def step(j, carry):
        pltpu.sync_copy(ones_v, cacc.at[dst_v.at[j]], add=True)
        return carry

    lax.fori_loop(0, STEPS, step, 0)
    plsc.subcore_barrier()
    pltpu.sync_copy(cacc.at[pl.ds(s * RPT, RPT)],
                    out_hbm.at[c, pl.ds(s * RPT, RPT)])


_cnt_kernel = pl.kernel(
    _cnt_body,
    out_type=jax.ShapeDtypeStruct((NC, NP, 16), jnp.float32),
    mesh=_MESH,
    scratch_types=[
        pltpu.VMEM((STEPS, K), jnp.int32),
        pltpu.VMEM((K, 16), jnp.float32),
        pltpu.VMEM_SHARED((NP, 16), jnp.float32),
    ],
    compiler_params=pltpu.CompilerParams(use_tc_tiling_on_sc=False),
)


def _edge_body(y_hbm, src3_hbm, dst3_hbm, zrows_hbm, out_hbm, zacc, src_v,
               dst_v, rows_a, rows_b, sem_a, sem_b):
    c = lax.axis_index("c")
    s = lax.axis_index("s")
    wid = c * NS + s
    pltpu.sync_copy(zrows_hbm, zacc.at[pl.ds(s * RPT, RPT)])
    pltpu.sync_copy(src3_hbm.at[wid], src_v)
    pltpu.sync_copy(dst3_hbm.at[wid], dst_v)
    plsc.subcore_barrier()

    def gather(j, buf, sem):
        return pltpu.async_copy(y_hbm.at[src_v.at[j]], buf, sem)

    def scatter(j, buf):
        pltpu.sync_copy(buf, zacc.at[dst_v.at[j]], add=True)

    gather(0, rows_a, sem_a)

    def step(i, carry):
        j = 2 * i
        gather(j + 1, rows_b, sem_b)
        pltpu.make_async_copy(y_hbm.at[src_v.at[j]], rows_a, sem_a).wait()
        scatter(j, rows_a)
        gather(j + 2, rows_a, sem_a)
        pltpu.make_async_copy(y_hbm.at[src_v.at[j + 1]], rows_b, sem_b).wait()
        scatter(j + 1, rows_b)
        return carry

    lax.fori_loop(0, (STEPS - 1) // 2, step, 0)
    pltpu.make_async_copy(y_hbm.at[src_v.at[STEPS - 1]], rows_a, sem_a).wait()
    scatter(STEPS - 1, rows_a)
    plsc.subcore_barrier()
    pltpu.sync_copy(zacc.at[pl.ds(s * RPT, RPT)],
                    out_hbm.at[c, pl.ds(s * RPT, RPT)])


_edge_kernel = pl.kernel(
    _edge_body,
    out_type=jax.ShapeDtypeStruct((NC, NP, D), jnp.float32),
    mesh=_MESH,
    scratch_types=[
        pltpu.VMEM_SHARED((NP, D), jnp.float32),
        pltpu.VMEM((STEPS, K), jnp.int32),
        pltpu.VMEM((STEPS, K), jnp.int32),
        pltpu.VMEM((K, D), jnp.float32),
        pltpu.VMEM((K, D), jnp.float32),
        pltpu.SemaphoreType.DMA,
        pltpu.SemaphoreType.DMA,
    ],
    compiler_params=pltpu.CompilerParams(use_tc_tiling_on_sc=False),
)


BN = 1000


def _dinv(cnt_ref):
    deg = cnt_ref[0, :, 0:1] + cnt_ref[1, :, 0:1] + 1.0
    return lax.rsqrt(deg)


def _k1_body(cnt_ref, x_ref, w_ref, y_ref):
    y_ref[...] = _dinv(cnt_ref) * jnp.dot(
        x_ref[...], w_ref[...], preferred_element_type=jnp.float32)


def _k2_body(cnt_ref, z_ref, y_ref, b_ref, w_ref, o_ref):
    dinv = _dinv(cnt_ref)
    h = jnp.maximum(
        dinv * (z_ref[0] + z_ref[1] + y_ref[...]) + b_ref[...], 0.0)
    o_ref[...] = dinv * jnp.dot(
        h, w_ref[...], preferred_element_type=jnp.float32)


def _k3_body(cnt_ref, z_ref, y_ref, b_ref, o_ref):
    o_ref[...] = (_dinv(cnt_ref) * (z_ref[0] + z_ref[1] + y_ref[...])
                  + b_ref[...])


_cnt_spec = pl.BlockSpec((NC, BN, 16), lambda i: (0, i, 0))
_row_spec = pl.BlockSpec((BN, D), lambda i: (i, 0))
_z_spec = pl.BlockSpec((NC, BN, D), lambda i: (0, i, 0))
_w_spec = pl.BlockSpec((D, D), lambda i: (0, 0))
_b_spec = pl.BlockSpec((1, D), lambda i: (0, 0))
_out_shape = jax.ShapeDtypeStruct((N, D), jnp.float32)

_k1 = pl.pallas_call(
    _k1_body, grid=(N // BN,),
    in_specs=[_cnt_spec, _row_spec, _w_spec],
    out_specs=_row_spec, out_shape=_out_shape)

_k2 = pl.pallas_call(
    _k2_body, grid=(N // BN,),
    in_specs=[_cnt_spec, _z_spec, _row_spec, _b_spec, _w_spec],
    out_specs=_row_spec, out_shape=_out_shape)

_k3 = pl.pallas_call(
    _k3_body, grid=(N // BN,),
    in_specs=[_cnt_spec, _z_spec, _row_spec, _b_spec],
    out_specs=_row_spec, out_shape=_out_shape)


def kernel(x, edge_index, W1, b1, W2, b2):
    src3 = edge_index[0].reshape(NW, STEPS, K)
    dst3 = edge_index[1].reshape(NW, STEPS, K)
    ones16 = jnp.ones((K, 16), jnp.float32)
    z16 = jnp.zeros((RPT, 16), jnp.float32)
    zrows = jnp.zeros((RPT, D), jnp.float32)

    cnt = _cnt_kernel(dst3, ones16, z16)
    y1 = _k1(cnt, x, W1)
    z1 = _edge_kernel(y1, src3, dst3, zrows)
    y2 = _k2(cnt, z1, y1, b1.reshape(1, D), W2)
    z2 = _edge_kernel(y2, src3, dst3, zrows)
    return _k3(cnt, z2, y2, b2.reshape(1, D))


# speedup vs baseline: 33.4338x; 1.1464x over previous
import jax
import jax.numpy as jnp
from jax import lax
from jax.experimental import pallas as pl
from jax.experimental.pallas import tpu as pltpu
from jax.experimental.pallas import tpu_sc as plsc

N = 10000
E = 320000
D = 128

NC = 2
NS = 16
NW = NC * NS
K = 40
NBUF = 5
STEPS = E // (NW * K)
NP = 10240
RPT = NP // NS

_MESH = plsc.VectorSubcoreMesh(core_axis_name="c", subcore_axis_name="s")


def _cnt_body(dst3_hbm, ones_hbm, z16_hbm, out_hbm, dst_v, ones_v, cacc):
    c = lax.axis_index("c")
    s = lax.axis_index("s")
    wid = c * NS + s
    pltpu.sync_copy(z16_hbm, cacc.at[pl.ds(s * RPT, RPT)])
    pltpu.sync_copy(ones_hbm, ones_v)
    pltpu.sync_copy(dst3_hbm.at[wid], dst_v)
    plsc.subcore_barrier()

    def step(j, carry):
        pltpu.sync_copy(ones_v, cacc.at[dst_v.at[j]], add=True)
        return carry

    lax.fori_loop(0, STEPS, step, 0)
    plsc.subcore_barrier()
    pltpu.sync_copy(cacc.at[pl.ds(s * RPT, RPT)],
                    out_hbm.at[c, pl.ds(s * RPT, RPT)])


_cnt_kernel = pl.kernel(
    _cnt_body,
    out_type=jax.ShapeDtypeStruct((NC, NP, 16), jnp.float32),
    mesh=_MESH,
    scratch_types=[
        pltpu.VMEM((STEPS, K), jnp.int32),
        pltpu.VMEM((K, 16), jnp.float32),
        pltpu.VMEM_SHARED((NP, 16), jnp.float32),
    ],
    compiler_params=pltpu.CompilerParams(use_tc_tiling_on_sc=False),
)


def _edge_body(y_hbm, src3_hbm, dst3_hbm, zrows_hbm, out_hbm, zacc, src_v,
               dst_v, rows, gsem, ssem):
    c = lax.axis_index("c")
    s = lax.axis_index("s")
    wid = c * NS + s
    pltpu.sync_copy(zrows_hbm, zacc.at[pl.ds(s * RPT, RPT)])
    pltpu.sync_copy(src3_hbm.at[wid], src_v)
    pltpu.sync_copy(dst3_hbm.at[wid], dst_v)
    plsc.subcore_barrier()

    def gather_start(j, b):
        pltpu.async_copy(y_hbm.at[src_v.at[j]], rows[b], gsem[b])

    def gather_wait(j, b):
        pltpu.make_async_copy(y_hbm.at[src_v.at[j]], rows[b], gsem[b]).wait()

    def scatter_start(j, b):
        pltpu.async_copy(rows[b], zacc.at[dst_v.at[j]], ssem[b], add=True)

    def scatter_wait(j, b):
        pltpu.make_async_copy(rows[b], zacc.at[dst_v.at[j]], ssem[b]).wait()

    for b in range(NBUF):
        gather_start(b, b)

    def step(i, carry):
        j0 = i * NBUF
        for b in range(NBUF):
            j = j0 + b
            gather_wait(j, b)
            scatter_start(j, b)

            @pl.when(j + NBUF < STEPS)
            def _():
                scatter_wait(j, b)
                gather_start(j + NBUF, b)

            del _
        return carry

    lax.fori_loop(0, STEPS // NBUF, step, 0)
    for b in range(NBUF):
        scatter_wait(STEPS - NBUF + b, b)
    plsc.subcore_barrier()
    pltpu.sync_copy(zacc.at[pl.ds(s * RPT, RPT)],
                    out_hbm.at[c, pl.ds(s * RPT, RPT)])


_edge_kernel = pl.kernel(
    _edge_body,
    out_type=jax.ShapeDtypeStruct((NC, NP, D), jnp.float32),
    mesh=_MESH,
    scratch_types=[
        pltpu.VMEM_SHARED((NP, D), jnp.float32),
        pltpu.VMEM((STEPS, K), jnp.int32),
        pltpu.VMEM((STEPS, K), jnp.int32),
        [pltpu.VMEM((K, D), jnp.float32)] * NBUF,
        [pltpu.SemaphoreType.DMA] * NBUF,
        [pltpu.SemaphoreType.DMA] * NBUF,
    ],
    compiler_params=pltpu.CompilerParams(use_tc_tiling_on_sc=False),
)


BN = 1000


def _dinv(cnt_ref):
    deg = cnt_ref[0, :, 0:1] + cnt_ref[1, :, 0:1] + 1.0
    return lax.rsqrt(deg)


def _k1_body(cnt_ref, x_ref, w_ref, y_ref):
    y_ref[...] = _dinv(cnt_ref) * jnp.dot(
        x_ref[...], w_ref[...], preferred_element_type=jnp.float32)


def _k2_body(cnt_ref, z_ref, y_ref, b_ref, w_ref, o_ref):
    dinv = _dinv(cnt_ref)
    h = jnp.maximum(
        dinv * (z_ref[0] + z_ref[1] + y_ref[...]) + b_ref[...], 0.0)
    o_ref[...] = dinv * jnp.dot(
        h, w_ref[...], preferred_element_type=jnp.float32)


def _k3_body(cnt_ref, z_ref, y_ref, b_ref, o_ref):
    o_ref[...] = (_dinv(cnt_ref) * (z_ref[0] + z_ref[1] + y_ref[...])
                  + b_ref[...])


_cnt_spec = pl.BlockSpec((NC, BN, 16), lambda i: (0, i, 0))
_row_spec = pl.BlockSpec((BN, D), lambda i: (i, 0))
_z_spec = pl.BlockSpec((NC, BN, D), lambda i: (0, i, 0))
_w_spec = pl.BlockSpec((D, D), lambda i: (0, 0))
_b_spec = pl.BlockSpec((1, D), lambda i: (0, 0))
_out_shape = jax.ShapeDtypeStruct((N, D), jnp.float32)

_k1 = pl.pallas_call(
    _k1_body, grid=(N // BN,),
    in_specs=[_cnt_spec, _row_spec, _w_spec],
    out_specs=_row_spec, out_shape=_out_shape)

_k2 = pl.pallas_call(
    _k2_body, grid=(N // BN,),
    in_specs=[_cnt_spec, _z_spec, _row_spec, _b_spec, _w_spec],
    out_specs=_row_spec, out_shape=_out_shape)

_k3 = pl.pallas_call(
    _k3_body, grid=(N // BN,),
    in_specs=[_cnt_spec, _z_spec, _row_spec, _b_spec],
    out_specs=_row_spec, out_shape=_out_shape)


def kernel(x, edge_index, W1, b1, W2, b2):
    src3 = edge_index[0].reshape(NW, STEPS, K)
    dst3 = edge_index[1].reshape(NW, STEPS, K)
    ones16 = jnp.ones((K, 16), jnp.float32)
    z16 = jnp.zeros((RPT, 16), jnp.float32)
    zrows = jnp.zeros((RPT, D), jnp.float32)

    cnt = _cnt_kernel(dst3, ones16, z16)
    y1 = _k1(cnt, x, W1)
    z1 = _edge_kernel(y1, src3, dst3, zrows)
    y2 = _k2(cnt, z1, y1, b1.reshape(1, D), W2)
    z2 = _edge_kernel(y2, src3, dst3, zrows)
    return _k3(cnt, z2, y2, b2.reshape(1, D))
